# baseline (speedup 1.0000x reference)
import functools

import jax
import jax.numpy as jnp
from jax import lax
from jax.experimental import pallas as pl
from jax.experimental.pallas import tpu as pltpu
from jax.experimental.pallas import tpu_sc as plsc

VOCAB = 1000000
EMBED = 16
N_IDX = 16384

NC = 2
NS = 16
NW = NC * NS

IDX_ROW = 128
ROWS_PER_W = N_IDX // NW
CHUNKS = ROWS_PER_W // IDX_ROW
UNROLL = 8

PACK = 128 // EMBED
_G = 125
_RPB = VOCAB // PACK // _G


@functools.cache
def _sc_gather_sum():
    mesh = plsc.VectorSubcoreMesh(
        core_axis_name="c", subcore_axis_name="s", num_cores=NC, num_subcores=NS
    )

    @functools.partial(
        pl.kernel,
        out_type=jax.ShapeDtypeStruct((NW, EMBED), jnp.float32),
        mesh=mesh,
        scratch_types=[
            pltpu.VMEM((CHUNKS, IDX_ROW), jnp.int32),
            pltpu.VMEM((ROWS_PER_W, EMBED), jnp.float32),
            pltpu.VMEM((EMBED,), jnp.float32),
            pltpu.SemaphoreType.DMA,
        ],
        compiler_params=pltpu.CompilerParams(use_tc_tiling_on_sc=False),
    )
    def sc_body(idx_hbm, table_hbm, out_hbm, idx_v, rows_v, acc_v, sem):
        wid = lax.axis_index("s") * NC + lax.axis_index("c")
        pltpu.sync_copy(idx_hbm.at[pl.ds(wid * CHUNKS, CHUNKS)], idx_v)
        copies = []
        for j in range(CHUNKS):
            copies.append(
                pltpu.async_copy(
                    table_hbm.at[idx_v.at[j]],
                    rows_v.at[pl.ds(j * IDX_ROW, IDX_ROW)],
                    sem,
                )
            )
        for c in copies:
            c.wait()

        def body(r, accs):
            base = r * UNROLL
            return tuple(
                a + rows_v[base + k, :] for k, a in enumerate(accs)
            )

        zero = jnp.zeros((EMBED,), jnp.float32)
        accs = lax.fori_loop(
            0, ROWS_PER_W // UNROLL, body, (zero,) * UNROLL
        )
        acc = functools.reduce(lambda a, b: a + b, accs)
        acc_v[...] = acc
        pltpu.sync_copy(acc_v, out_hbm.at[wid])

    return sc_body


def _tc_matvec_body(p_ref, w_ref, b_ref, o_ref):
    emb = p_ref[...].sum(axis=0) * (1.0 / N_IDX)
    eye = jnp.eye(EMBED, dtype=jnp.float32)
    emb_col = (jnp.broadcast_to(emb, (EMBED, EMBED)) * eye).sum(
        axis=1, keepdims=True
    )
    col = jnp.concatenate([emb_col] * PACK, axis=0)
    row_grp = lax.broadcasted_iota(jnp.int32, (PACK * EMBED, PACK), 0) // EMBED
    col_id = lax.broadcasted_iota(jnp.int32, (PACK * EMBED, PACK), 1)
    r_mat = jnp.where(row_grp == col_id, col, 0.0)
    y = jax.lax.dot_general(
        w_ref[0], r_mat, (((1,), (0,)), ((), ())),
        preferred_element_type=jnp.float32,
    )
    o_ref[0] = y + b_ref[0]


def kernel(inputs, emb_table, W, b):
    idx2d = inputs.astype(jnp.int32).reshape(NW * CHUNKS, IDX_ROW)
    partials = _sc_gather_sum()(idx2d, emb_table)

    w3 = W.reshape(_G, _RPB, PACK * EMBED)
    b3 = b.reshape(_G, _RPB, PACK)
    out = pl.pallas_call(
        _tc_matvec_body,
        grid=(_G,),
        in_specs=[
            pl.BlockSpec((NW, EMBED), lambda i: (0, 0)),
            pl.BlockSpec((1, _RPB, PACK * EMBED), lambda i: (i, 0, 0)),
            pl.BlockSpec((1, _RPB, PACK), lambda i: (i, 0, 0)),
        ],
        out_specs=pl.BlockSpec((1, _RPB, PACK), lambda i: (i, 0, 0)),
        out_shape=jax.ShapeDtypeStruct((_G, _RPB, PACK), jnp.float32),
        compiler_params=pltpu.CompilerParams(
            dimension_semantics=("arbitrary",),
        ),
    )(partials, w3, b3)
    return out.reshape(VOCAB)


# speedup vs baseline: 8.9808x; 8.9808x over previous
import functools

import jax
import jax.numpy as jnp
from jax import lax
from jax.experimental import pallas as pl
from jax.experimental.pallas import tpu as pltpu

VOCAB = 1000000
EMBED = 16
N_IDX = 16384

BV = 8192
GRID = (VOCAB + BV - 1) // BV


def _tc_matvec_body(e_ref, wt_ref, b_ref, o_ref):
    emb_row = e_ref[...]
    y = jax.lax.dot_general(
        emb_row, wt_ref[...], (((1,), (0,)), ((), ())),
        preferred_element_type=jnp.float32,
    )
    o_ref[...] = jnp.reshape(y, (BV,)) + b_ref[...]


def kernel(inputs, emb_table, W, b):
    embeds = jnp.take(emb_table, inputs, axis=0).mean(axis=0)
    wt = W.T
    out = pl.pallas_call(
        _tc_matvec_body,
        grid=(GRID,),
        in_specs=[
            pl.BlockSpec((1, EMBED), lambda i: (0, 0)),
            pl.BlockSpec((EMBED, BV), lambda i: (0, i)),
            pl.BlockSpec((BV,), lambda i: (i,)),
        ],
        out_specs=pl.BlockSpec((BV,), lambda i: (i,)),
        out_shape=jax.ShapeDtypeStruct((VOCAB,), jnp.float32),
        compiler_params=pltpu.CompilerParams(
            dimension_semantics=("arbitrary",),
        ),
    )(embeds.reshape(1, EMBED), wt, b)
    return out


# speedup vs baseline: 15.4199x; 1.7170x over previous
import functools

import jax
import jax.numpy as jnp
from jax import lax
from jax.experimental import pallas as pl
from jax.experimental.pallas import tpu as pltpu

VOCAB = 1000000
EMBED = 16
N_IDX = 16384

BV = 32768
GRID = (VOCAB + BV - 1) // BV


def _tc_matvec_body(e_ref, wt_ref, b_ref, o_ref):
    emb_row = e_ref[...]
    y = jax.lax.dot_general(
        emb_row, wt_ref[...], (((1,), (0,)), ((), ())),
        preferred_element_type=jnp.float32,
    )
    o_ref[...] = jnp.reshape(y, (BV,)) + b_ref[...]


def kernel(inputs, emb_table, W, b):
    embeds = jnp.take(emb_table, inputs, axis=0).mean(axis=0)
    wt = W.T
    out = pl.pallas_call(
        _tc_matvec_body,
        grid=(GRID,),
        in_specs=[
            pl.BlockSpec((1, EMBED), lambda i: (0, 0)),
            pl.BlockSpec((EMBED, BV), lambda i: (0, i)),
            pl.BlockSpec((BV,), lambda i: (i,)),
        ],
        out_specs=pl.BlockSpec((BV,), lambda i: (i,)),
        out_shape=jax.ShapeDtypeStruct((VOCAB,), jnp.float32),
        compiler_params=pltpu.CompilerParams(
            dimension_semantics=("arbitrary",),
        ),
    )(embeds.reshape(1, EMBED), wt, b)
    return out


# speedup vs baseline: 17.7097x; 1.1485x over previous
import functools

import jax
import jax.numpy as jnp
from jax import lax
from jax.experimental import pallas as pl
from jax.experimental.pallas import tpu as pltpu

VOCAB = 1000000
EMBED = 16
N_IDX = 16384

BV = 65536
GRID = (VOCAB + BV - 1) // BV


def _tc_matvec_body(e_ref, wt_ref, b_ref, o_ref):
    emb_row = e_ref[...]
    y = jax.lax.dot_general(
        emb_row, wt_ref[...], (((1,), (0,)), ((), ())),
        preferred_element_type=jnp.float32,
    )
    o_ref[...] = jnp.reshape(y, (BV,)) + b_ref[...]


def kernel(inputs, emb_table, W, b):
    embeds = jnp.take(emb_table, inputs, axis=0).mean(axis=0)
    wt = W.T
    out = pl.pallas_call(
        _tc_matvec_body,
        grid=(GRID,),
        in_specs=[
            pl.BlockSpec((1, EMBED), lambda i: (0, 0)),
            pl.BlockSpec((EMBED, BV), lambda i: (0, i)),
            pl.BlockSpec((BV,), lambda i: (i,)),
        ],
        out_specs=pl.BlockSpec((BV,), lambda i: (i,)),
        out_shape=jax.ShapeDtypeStruct((VOCAB,), jnp.float32),
        compiler_params=pltpu.CompilerParams(
            dimension_semantics=("arbitrary",),
        ),
    )(embeds.reshape(1, EMBED), wt, b)
    return out


# speedup vs baseline: 18.2434x; 1.0301x over previous
import functools

import jax
import jax.numpy as jnp
from jax import lax
from jax.experimental import pallas as pl
from jax.experimental.pallas import tpu as pltpu

VOCAB = 1000000
EMBED = 16
N_IDX = 16384

BV = 131072
GRID = (VOCAB + BV - 1) // BV


def _tc_matvec_body(e_ref, wt_ref, b_ref, o_ref):
    emb_row = e_ref[...]
    y = jax.lax.dot_general(
        emb_row, wt_ref[...], (((1,), (0,)), ((), ())),
        preferred_element_type=jnp.float32,
    )
    o_ref[...] = jnp.reshape(y, (BV,)) + b_ref[...]


def kernel(inputs, emb_table, W, b):
    embeds = jnp.take(emb_table, inputs, axis=0).mean(axis=0)
    wt = W.T
    out = pl.pallas_call(
        _tc_matvec_body,
        grid=(GRID,),
        in_specs=[
            pl.BlockSpec((1, EMBED), lambda i: (0, 0)),
            pl.BlockSpec((EMBED, BV), lambda i: (0, i)),
            pl.BlockSpec((BV,), lambda i: (i,)),
        ],
        out_specs=pl.BlockSpec((BV,), lambda i: (i,)),
        out_shape=jax.ShapeDtypeStruct((VOCAB,), jnp.float32),
        compiler_params=pltpu.CompilerParams(
            dimension_semantics=("arbitrary",),
        ),
    )(embeds.reshape(1, EMBED), wt, b)
    return out
